# baseline (speedup 1.0000x reference)
import functools

import jax
import jax.numpy as jnp
from jax import lax
from jax.experimental import pallas as pl
from jax.experimental.pallas import tpu as pltpu
from jax.experimental.pallas import tpu_sc as plsc

B = 16384
D = 16
PACK = 8
W = D * PACK
NCOL = 6
_NC = 2
_NS = 16
_NW = _NC * _NS
_BPW = B // _NW
_CH = 256
_NCH = _BPW // _CH


def _sc_gather(tables8, row_hi):
  mesh = plsc.VectorSubcoreMesh(core_axis_name="c", subcore_axis_name="s")

  @functools.partial(
      pl.kernel,
      mesh=mesh,
      compiler_params=pltpu.CompilerParams(use_tc_tiling_on_sc=True),
      out_type=tuple(
          jax.ShapeDtypeStruct((B, W), jnp.float32) for _ in range(NCOL)),
      scratch_types=(
          [pltpu.VMEM((_BPW,), jnp.int32) for _ in range(NCOL)]
          + [pltpu.VMEM((_CH, W), jnp.float32) for _ in range(2)]
          + [pltpu.SemaphoreType.DMA for _ in range(2)]
      ),
  )
  def k(*refs):
    tabs = refs[0:NCOL]
    idxs = refs[NCOL:2 * NCOL]
    outs = refs[2 * NCOL:3 * NCOL]
    idx_v = refs[3 * NCOL:4 * NCOL]
    bufs = refs[4 * NCOL:4 * NCOL + 2]
    sems = refs[4 * NCOL + 2:4 * NCOL + 4]
    wid = lax.axis_index("s") * _NC + lax.axis_index("c")
    base = wid * _BPW
    for j in range(NCOL):
      pltpu.sync_copy(idxs[j].at[pl.ds(base, _BPW)], idx_v[j])

    units = [(j, h) for j in range(NCOL) for h in range(_NCH)]

    def start(u):
      j, h = units[u]
      s = u % 2
      return pltpu.async_copy(
          tabs[j].at[idx_v[j].at[pl.ds(h * _CH, _CH)]], bufs[s], sems[s])

    cp = start(0)
    for u in range(len(units)):
      cp.wait()
      nxt = start(u + 1) if u + 1 < len(units) else None
      j, h = units[u]
      pltpu.sync_copy(bufs[u % 2], outs[j].at[pl.ds(base + h * _CH, _CH)])
      cp = nxt

  return k(*tables8, *row_hi)


_R = 2048


def _mlp_body(g0, g1, g2, g3, g4, g5, o0, o1, o2, o3, o4, o5,
              w0e, b0, w1, b1, w2, b2, out):
  gs = (g0, g1, g2, g3, g4, g5)
  os_ = (o0, o1, o2, o3, o4, o5)
  lane_grp = jax.lax.broadcasted_iota(jnp.int32, (_R, W), 1) // D
  w0v = w0e[...]
  s = None
  for j in range(NCOL):
    m = lane_grp == os_[j][...]
    g = jnp.where(m, gs[j][...], 0.0)
    p = jnp.dot(g, w0v[W * j:W * (j + 1), :],
                preferred_element_type=jnp.float32)
    s = p if s is None else s + p
  h0 = jnp.maximum(s + b0[...], 0.0)
  h1 = jnp.maximum(
      jnp.dot(h0, w1[...], preferred_element_type=jnp.float32) + b1[...], 0.0)
  out[...] = jnp.dot(h1, w2[...], preferred_element_type=jnp.float32) + b2[...]


def _tc_mlp(packed, offs, W0, b0, W1, b1, W2, b2):
  gspec = pl.BlockSpec((_R, W), lambda g: (g, 0))
  ospec = pl.BlockSpec((_R, 1), lambda g: (g, 0))

  def wspec(shape):
    return pl.BlockSpec(shape, lambda g: (0, 0))

  w0e = jnp.concatenate(
      [jnp.tile(W0[D * j:D * (j + 1), :], (PACK, 1)) for j in range(NCOL)],
      axis=0)

  return pl.pallas_call(
      _mlp_body,
      grid=(B // _R,),
      in_specs=(
          [gspec] * NCOL + [ospec] * NCOL
          + [wspec((W * NCOL, 256)), wspec((1, 256)),
             wspec((256, 128)), wspec((1, 128)),
             wspec((128, 1)), wspec((1, 1))]
      ),
      out_specs=pl.BlockSpec((_R, 1), lambda g: (g, 0)),
      out_shape=jax.ShapeDtypeStruct((B, 1), jnp.float32),
  )(*packed, *offs, w0e, b0.reshape(1, -1), W1, b1.reshape(1, -1), W2,
    b2.reshape(1, -1))


def kernel(idx_user_id, table_user_id, idx_item_id, table_item_id,
           idx_device, table_device, idx_geo, table_geo,
           idx_hour, table_hour, idx_dayofweek, table_dayofweek,
           W0, b0, W1, b1, W2, b2):
  tables = [table_user_id, table_item_id, table_device, table_geo,
            table_hour, table_dayofweek]
  idxs = [idx_user_id, idx_item_id, idx_device, idx_geo, idx_hour,
          idx_dayofweek]
  idxs = [i.astype(jnp.int32) for i in idxs]
  t8 = []
  for t in tables:
    v = t.shape[0]
    if v % PACK:
      t = jnp.pad(t, ((0, PACK - v % PACK), (0, 0)))
    t8.append(jnp.reshape(t, (t.shape[0] // PACK, W)))
  row_hi = [i // PACK for i in idxs]
  offs = [(i % PACK).reshape(B, 1) for i in idxs]
  packed = _sc_gather(t8, row_hi)
  return _tc_mlp(packed, offs, W0, b0, W1, b1, W2, b2)


# speedup vs baseline: 1.5857x; 1.5857x over previous
import functools

import jax
import jax.numpy as jnp
from jax import lax
from jax.experimental import pallas as pl
from jax.experimental.pallas import tpu as pltpu
from jax.experimental.pallas import tpu_sc as plsc

B = 16384
D = 16
NCOL = 6
_NC = 2
_NS = 16
_NW = _NC * _NS
_BPW = B // _NW
_BF = (True, True, False, False, False, False)


def _sc_gather(tables, indices):
  mesh = plsc.VectorSubcoreMesh(core_axis_name="c", subcore_axis_name="s")

  @functools.partial(
      pl.kernel,
      mesh=mesh,
      compiler_params=pltpu.CompilerParams(use_tc_tiling_on_sc=False),
      out_type=tuple(
          jax.ShapeDtypeStruct((B, D), jnp.bfloat16 if _BF[j] else jnp.float32)
          for j in range(NCOL)),
      scratch_types=(
          [pltpu.VMEM((_BPW,), jnp.int32) for _ in range(NCOL)]
          + [pltpu.VMEM((_BPW, D), jnp.bfloat16 if _BF[j] else jnp.float32)
             for j in range(NCOL)]
          + [pltpu.SemaphoreType.DMA for _ in range(NCOL)]
      ),
  )
  def k(*refs):
    tabs = refs[0:NCOL]
    idxs = refs[NCOL:2 * NCOL]
    outs = refs[2 * NCOL:3 * NCOL]
    idx_v = refs[3 * NCOL:4 * NCOL]
    rows_v = refs[4 * NCOL:5 * NCOL]
    sems = refs[5 * NCOL:6 * NCOL]
    wid = lax.axis_index("s") * _NC + lax.axis_index("c")
    base = wid * _BPW
    for j in range(NCOL):
      pltpu.sync_copy(idxs[j].at[pl.ds(base, _BPW)], idx_v[j])
    cps = [
        pltpu.async_copy(tabs[j].at[idx_v[j]], rows_v[j], sems[j])
        for j in range(NCOL)
    ]
    for j in range(NCOL):
      cps[j].wait()
      pltpu.sync_copy(rows_v[j], outs[j].at[pl.ds(base, _BPW)])

  return k(*tables, *indices)


_R = 2048


def _mlp_body(e0, e1, e2, e3, e4, e5, w0, b0, w1, b1, w2, b2, out):
  es = (e0, e1, e2, e3, e4, e5)
  w0v = w0[...]
  s = None
  for j in range(NCOL):
    x = es[j][...].astype(jnp.float32)
    p = jnp.dot(x, w0v[D * j:D * (j + 1), :],
                preferred_element_type=jnp.float32)
    s = p if s is None else s + p
  h0 = jnp.maximum(s + b0[...], 0.0)
  h1 = jnp.maximum(
      jnp.dot(h0, w1[...], preferred_element_type=jnp.float32) + b1[...], 0.0)
  out[...] = jnp.dot(h1, w2[...], preferred_element_type=jnp.float32) + b2[...]


def _tc_mlp(embs, W0, b0, W1, b1, W2, b2):
  especs = [pl.BlockSpec((_R, D), lambda g: (g, 0)) for _ in range(NCOL)]

  def wspec(shape):
    return pl.BlockSpec(shape, lambda g: (0, 0))

  return pl.pallas_call(
      _mlp_body,
      grid=(B // _R,),
      in_specs=(
          especs
          + [wspec((D * NCOL, 256)), wspec((1, 256)),
             wspec((256, 128)), wspec((1, 128)),
             wspec((128, 1)), wspec((1, 1))]
      ),
      out_specs=pl.BlockSpec((_R, 1), lambda g: (g, 0)),
      out_shape=jax.ShapeDtypeStruct((B, 1), jnp.float32),
  )(*embs, W0, b0.reshape(1, -1), W1, b1.reshape(1, -1), W2,
    b2.reshape(1, -1))


def kernel(idx_user_id, table_user_id, idx_item_id, table_item_id,
           idx_device, table_device, idx_geo, table_geo,
           idx_hour, table_hour, idx_dayofweek, table_dayofweek,
           W0, b0, W1, b1, W2, b2):
  tables = [table_user_id, table_item_id, table_device, table_geo,
            table_hour, table_dayofweek]
  tables = [t.astype(jnp.bfloat16) if _BF[j] else t
            for j, t in enumerate(tables)]
  indices = tuple(
      i.astype(jnp.int32)
      for i in (idx_user_id, idx_item_id, idx_device, idx_geo, idx_hour,
                idx_dayofweek))
  embs = _sc_gather(tables, indices)
  return _tc_mlp(embs, W0, b0, W1, b1, W2, b2)


# speedup vs baseline: 2.3707x; 1.4951x over previous
import functools

import jax
import jax.numpy as jnp
from jax import lax
from jax.experimental import pallas as pl
from jax.experimental.pallas import tpu as pltpu
from jax.experimental.pallas import tpu_sc as plsc

B = 16384
D = 16
NCOL = 6
_NC = 2
_NS = 16
_NW = _NC * _NS
_BPW = B // _NW
_BF = (False, False, False, False, False, False)


def _sc_gather(tables, indices):
  mesh = plsc.VectorSubcoreMesh(core_axis_name="c", subcore_axis_name="s")

  @functools.partial(
      pl.kernel,
      mesh=mesh,
      compiler_params=pltpu.CompilerParams(use_tc_tiling_on_sc=False),
      out_type=tuple(
          jax.ShapeDtypeStruct((B, D), jnp.bfloat16 if _BF[j] else jnp.float32)
          for j in range(NCOL)),
      scratch_types=(
          [pltpu.VMEM((_BPW,), jnp.int32) for _ in range(NCOL)]
          + [pltpu.VMEM((_BPW, D), jnp.bfloat16 if _BF[j] else jnp.float32)
             for j in range(NCOL)]
          + [pltpu.SemaphoreType.DMA for _ in range(NCOL)]
      ),
  )
  def k(*refs):
    tabs = refs[0:NCOL]
    idxs = refs[NCOL:2 * NCOL]
    outs = refs[2 * NCOL:3 * NCOL]
    idx_v = refs[3 * NCOL:4 * NCOL]
    rows_v = refs[4 * NCOL:5 * NCOL]
    sems = refs[5 * NCOL:6 * NCOL]
    wid = lax.axis_index("s") * _NC + lax.axis_index("c")
    base = wid * _BPW
    for j in range(NCOL):
      pltpu.sync_copy(idxs[j].at[pl.ds(base, _BPW)], idx_v[j])
    cps = [
        pltpu.async_copy(tabs[j].at[idx_v[j]], rows_v[j], sems[j])
        for j in range(NCOL)
    ]
    for j in range(NCOL):
      cps[j].wait()
      pltpu.sync_copy(rows_v[j], outs[j].at[pl.ds(base, _BPW)])

  return k(*tables, *indices)


_V = 1000000
_TCOL = 2048
_TGRID = (_V + _TCOL - 1) // _TCOL
_KB = _TCOL // 8
_VPAD = _TGRID * _TCOL


def _tr_body(a, b, oa, ob):
  for x, o in ((a, oa), (b, ob)):
    t = x[...].T
    o[...] = jnp.concatenate(
        [t[s * _KB:(s + 1) * _KB, :] for s in range(8)], axis=1)


def _tc_transpose(t0, t1):
  ispec = pl.BlockSpec((16, _TCOL), lambda g: (0, g))
  ospec = pl.BlockSpec((_KB, 128), lambda g: (g, 0))
  return pl.pallas_call(
      _tr_body,
      grid=(_TGRID,),
      in_specs=[ispec, ispec],
      out_specs=[ospec, ospec],
      out_shape=[jax.ShapeDtypeStruct((_VPAD // 8, 128), jnp.float32)] * 2,
  )(t0, t1)


def _pack_idx(i):
  return (i & ~2047) | ((i & 255) << 3) | ((i >> 8) & 7)


_R = 2048


def _mlp_body(e0, e1, e2, e3, e4, e5, w0, b0, w1, b1, w2, b2, out):
  es = (e0, e1, e2, e3, e4, e5)
  w0v = w0[...]
  s = None
  for j in range(NCOL):
    x = es[j][...].astype(jnp.float32)
    p = jnp.dot(x, w0v[D * j:D * (j + 1), :],
                preferred_element_type=jnp.float32)
    s = p if s is None else s + p
  h0 = jnp.maximum(s + b0[...], 0.0)
  h1 = jnp.maximum(
      jnp.dot(h0, w1[...], preferred_element_type=jnp.float32) + b1[...], 0.0)
  out[...] = jnp.dot(h1, w2[...], preferred_element_type=jnp.float32) + b2[...]


def _tc_mlp(embs, W0, b0, W1, b1, W2, b2):
  especs = [pl.BlockSpec((_R, D), lambda g: (g, 0)) for _ in range(NCOL)]

  def wspec(shape):
    return pl.BlockSpec(shape, lambda g: (0, 0))

  return pl.pallas_call(
      _mlp_body,
      grid=(B // _R,),
      in_specs=(
          especs
          + [wspec((D * NCOL, 256)), wspec((1, 256)),
             wspec((256, 128)), wspec((1, 128)),
             wspec((128, 1)), wspec((1, 1))]
      ),
      out_specs=pl.BlockSpec((_R, 1), lambda g: (g, 0)),
      out_shape=jax.ShapeDtypeStruct((B, 1), jnp.float32),
  )(*embs, W0, b0.reshape(1, -1), W1, b1.reshape(1, -1), W2,
    b2.reshape(1, -1))


def kernel(idx_user_id, table_user_id, idx_item_id, table_item_id,
           idx_device, table_device, idx_geo, table_geo,
           idx_hour, table_hour, idx_dayofweek, table_dayofweek,
           W0, b0, W1, b1, W2, b2):
  p0, p1 = _tc_transpose(table_user_id.T, table_item_id.T)
  tables = [p0.reshape(_VPAD, D), p1.reshape(_VPAD, D), table_device,
            table_geo, table_hour, table_dayofweek]
  i0 = _pack_idx(idx_user_id.astype(jnp.int32))
  i1 = _pack_idx(idx_item_id.astype(jnp.int32))
  indices = (i0, i1) + tuple(
      i.astype(jnp.int32)
      for i in (idx_device, idx_geo, idx_hour, idx_dayofweek))
  embs = _sc_gather(tables, indices)
  return _tc_mlp(embs, W0, b0, W1, b1, W2, b2)


# speedup vs baseline: 3.7019x; 1.5615x over previous
import functools

import jax
import jax.numpy as jnp
from jax import lax
from jax.experimental import pallas as pl
from jax.experimental.pallas import tpu as pltpu
from jax.experimental.pallas import tpu_sc as plsc

B = 16384
D = 16
NCOL = 6
_NC = 2
_NS = 16
_NW = _NC * _NS
_BPW = B // _NW
_BF = (False, False, False, False, False, False)


def _sc_gather(tables, indices):
  mesh = plsc.VectorSubcoreMesh(core_axis_name="c", subcore_axis_name="s")

  @functools.partial(
      pl.kernel,
      mesh=mesh,
      compiler_params=pltpu.CompilerParams(use_tc_tiling_on_sc=False),
      out_type=tuple(
          jax.ShapeDtypeStruct((B, D), jnp.bfloat16 if _BF[j] else jnp.float32)
          for j in range(NCOL)),
      scratch_types=(
          [pltpu.VMEM((_BPW,), jnp.int32) for _ in range(NCOL)]
          + [pltpu.VMEM((_BPW, D), jnp.bfloat16 if _BF[j] else jnp.float32)
             for j in range(NCOL)]
          + [pltpu.SemaphoreType.DMA for _ in range(NCOL)]
      ),
  )
  def k(*refs):
    tabs = refs[0:NCOL]
    idxs = refs[NCOL:2 * NCOL]
    outs = refs[2 * NCOL:3 * NCOL]
    idx_v = refs[3 * NCOL:4 * NCOL]
    rows_v = refs[4 * NCOL:5 * NCOL]
    sems = refs[5 * NCOL:6 * NCOL]
    wid = lax.axis_index("s") * _NC + lax.axis_index("c")
    base = wid * _BPW
    for j in range(NCOL):
      pltpu.sync_copy(idxs[j].at[pl.ds(base, _BPW)], idx_v[j])
    cps = [
        pltpu.async_copy(tabs[j].at[idx_v[j]], rows_v[j], sems[j])
        for j in range(NCOL)
    ]
    for j in range(NCOL):
      cps[j].wait()
      pltpu.sync_copy(rows_v[j], outs[j].at[pl.ds(base, _BPW)])

  return k(*tables, *indices)


_V = 1000000
_TCOL = 8192
_TGRID = (_V + _TCOL - 1) // _TCOL
_KB = _TCOL // 8
_VPAD = _TGRID * _TCOL


def _tr_body(a, b, oa, ob):
  col = lax.broadcasted_iota(jnp.int32, (16, 128), 1)
  row = lax.broadcasted_iota(jnp.int32, (16, 128), 0)
  for x, o in ((a, oa), (b, ob)):
    xv = x[...]
    acc = None
    for s in range(8):
      e = (col == s * 16 + row).astype(jnp.float32)
      p = lax.dot_general(xv[:, s * _KB:(s + 1) * _KB], e,
                          (((0,), (0,)), ((), ())),
                          preferred_element_type=jnp.float32)
      acc = p if acc is None else acc + p
    o[...] = acc


def _tc_transpose(t0, t1):
  ispec = pl.BlockSpec((16, _TCOL), lambda g: (0, g))
  ospec = pl.BlockSpec((_KB, 128), lambda g: (g, 0))
  return pl.pallas_call(
      _tr_body,
      grid=(_TGRID,),
      in_specs=[ispec, ispec],
      out_specs=[ospec, ospec],
      out_shape=[jax.ShapeDtypeStruct((_VPAD // 8, 128), jnp.float32)] * 2,
      compiler_params=pltpu.CompilerParams(fuse_transposed_lhs_in_matmul=True),
  )(t0, t1)


def _pack_idx(i):
  r_local = i % _TCOL
  return (i - r_local) + (r_local % _KB) * 8 + r_local // _KB


_R = 2048


def _mlp_body(e0, e1, e2, e3, e4, e5, w0, b0, w1, b1, w2, b2, out):
  es = (e0, e1, e2, e3, e4, e5)
  w0v = w0[...]
  s = None
  for j in range(NCOL):
    x = es[j][...].astype(jnp.float32)
    p = jnp.dot(x, w0v[D * j:D * (j + 1), :],
                preferred_element_type=jnp.float32)
    s = p if s is None else s + p
  h0 = jnp.maximum(s + b0[...], 0.0)
  h1 = jnp.maximum(
      jnp.dot(h0, w1[...], preferred_element_type=jnp.float32) + b1[...], 0.0)
  out[...] = jnp.dot(h1, w2[...], preferred_element_type=jnp.float32) + b2[...]


def _tc_mlp(embs, W0, b0, W1, b1, W2, b2):
  especs = [pl.BlockSpec((_R, D), lambda g: (g, 0)) for _ in range(NCOL)]

  def wspec(shape):
    return pl.BlockSpec(shape, lambda g: (0, 0))

  return pl.pallas_call(
      _mlp_body,
      grid=(B // _R,),
      in_specs=(
          especs
          + [wspec((D * NCOL, 256)), wspec((1, 256)),
             wspec((256, 128)), wspec((1, 128)),
             wspec((128, 1)), wspec((1, 1))]
      ),
      out_specs=pl.BlockSpec((_R, 1), lambda g: (g, 0)),
      out_shape=jax.ShapeDtypeStruct((B, 1), jnp.float32),
  )(*embs, W0, b0.reshape(1, -1), W1, b1.reshape(1, -1), W2,
    b2.reshape(1, -1))


def kernel(idx_user_id, table_user_id, idx_item_id, table_item_id,
           idx_device, table_device, idx_geo, table_geo,
           idx_hour, table_hour, idx_dayofweek, table_dayofweek,
           W0, b0, W1, b1, W2, b2):
  p0, p1 = _tc_transpose(table_user_id.T, table_item_id.T)
  tables = [p0.reshape(_VPAD, D), p1.reshape(_VPAD, D), table_device,
            table_geo, table_hour, table_dayofweek]
  i0 = _pack_idx(idx_user_id.astype(jnp.int32))
  i1 = _pack_idx(idx_item_id.astype(jnp.int32))
  indices = (i0, i1) + tuple(
      i.astype(jnp.int32)
      for i in (idx_device, idx_geo, idx_hour, idx_dayofweek))
  embs = _sc_gather(tables, indices)
  return _tc_mlp(embs, W0, b0, W1, b1, W2, b2)


# speedup vs baseline: 6.3549x; 1.7167x over previous
import functools

import jax
import jax.numpy as jnp
from jax import lax
from jax.experimental import pallas as pl
from jax.experimental.pallas import tpu as pltpu
from jax.experimental.pallas import tpu_sc as plsc

B = 16384
D = 16
NCOL = 6
_NC = 2
_NS = 16
_NW = _NC * _NS
_BPW = B // _NW
_BF = (False, False, False, False, False, False)


def _sc_gather(tables, indices):
  mesh = plsc.VectorSubcoreMesh(core_axis_name="c", subcore_axis_name="s")

  @functools.partial(
      pl.kernel,
      mesh=mesh,
      compiler_params=pltpu.CompilerParams(use_tc_tiling_on_sc=False),
      out_type=tuple(
          jax.ShapeDtypeStruct((B, D), jnp.bfloat16 if _BF[j] else jnp.float32)
          for j in range(NCOL)),
      scratch_types=(
          [pltpu.VMEM((_BPW,), jnp.int32) for _ in range(NCOL)]
          + [pltpu.VMEM((_BPW, D), jnp.bfloat16 if _BF[j] else jnp.float32)
             for j in range(NCOL)]
          + [pltpu.SemaphoreType.DMA for _ in range(NCOL)]
      ),
  )
  def k(*refs):
    tabs = refs[0:NCOL]
    idxs = refs[NCOL:2 * NCOL]
    outs = refs[2 * NCOL:3 * NCOL]
    idx_v = refs[3 * NCOL:4 * NCOL]
    rows_v = refs[4 * NCOL:5 * NCOL]
    sems = refs[5 * NCOL:6 * NCOL]
    wid = lax.axis_index("s") * _NC + lax.axis_index("c")
    base = wid * _BPW
    for j in range(NCOL):
      pltpu.sync_copy(idxs[j].at[pl.ds(base, _BPW)], idx_v[j])
    cps = [
        pltpu.async_copy(tabs[j].at[idx_v[j]], rows_v[j], sems[j])
        for j in range(NCOL)
    ]
    for j in range(NCOL):
      cps[j].wait()
      pltpu.sync_copy(rows_v[j], outs[j].at[pl.ds(base, _BPW)])

  return k(*tables, *indices)


_V = 1000000
_TCOL = 16384
_TGRID = (_V + _TCOL - 1) // _TCOL
_KB = _TCOL // 8
_VPAD = _TGRID * _TCOL


def _tr_body(a, b, oa, ob):
  col = lax.broadcasted_iota(jnp.int32, (128, 128), 1)
  row = lax.broadcasted_iota(jnp.int32, (128, 128), 0)
  eye = (col == row).astype(jnp.float32)
  for x, o in ((a, oa), (b, ob)):
    xv = x[...]
    xc = jnp.concatenate(
        [xv[:, s * _KB:(s + 1) * _KB] for s in range(8)], axis=0)
    o[...] = lax.dot_general(xc, eye, (((0,), (0,)), ((), ())),
                             preferred_element_type=jnp.float32)


def _tc_transpose(t0, t1):
  ispec = pl.BlockSpec((16, _TCOL), lambda g: (0, g))
  ospec = pl.BlockSpec((_KB, 128), lambda g: (g, 0))
  return pl.pallas_call(
      _tr_body,
      grid=(_TGRID,),
      in_specs=[ispec, ispec],
      out_specs=[ospec, ospec],
      out_shape=[jax.ShapeDtypeStruct((_VPAD // 8, 128), jnp.float32)] * 2,
      compiler_params=pltpu.CompilerParams(fuse_transposed_lhs_in_matmul=True),
  )(t0, t1)


def _pack_idx(i):
  r_local = i % _TCOL
  return (i - r_local) + (r_local % _KB) * 8 + r_local // _KB


_R = 2048


def _mlp_body(e0, e1, e2, e3, e4, e5, w0, b0, w1, b1, w2, b2, out):
  es = (e0, e1, e2, e3, e4, e5)
  w0v = w0[...]
  s = None
  for j in range(NCOL):
    x = es[j][...].astype(jnp.float32)
    p = jnp.dot(x, w0v[D * j:D * (j + 1), :],
                preferred_element_type=jnp.float32)
    s = p if s is None else s + p
  h0 = jnp.maximum(s + b0[...], 0.0)
  h1 = jnp.maximum(
      jnp.dot(h0, w1[...], preferred_element_type=jnp.float32) + b1[...], 0.0)
  out[...] = jnp.dot(h1, w2[...], preferred_element_type=jnp.float32) + b2[...]


def _tc_mlp(embs, W0, b0, W1, b1, W2, b2):
  especs = [pl.BlockSpec((_R, D), lambda g: (g, 0)) for _ in range(NCOL)]

  def wspec(shape):
    return pl.BlockSpec(shape, lambda g: (0, 0))

  return pl.pallas_call(
      _mlp_body,
      grid=(B // _R,),
      in_specs=(
          especs
          + [wspec((D * NCOL, 256)), wspec((1, 256)),
             wspec((256, 128)), wspec((1, 128)),
             wspec((128, 1)), wspec((1, 1))]
      ),
      out_specs=pl.BlockSpec((_R, 1), lambda g: (g, 0)),
      out_shape=jax.ShapeDtypeStruct((B, 1), jnp.float32),
  )(*embs, W0, b0.reshape(1, -1), W1, b1.reshape(1, -1), W2,
    b2.reshape(1, -1))


def kernel(idx_user_id, table_user_id, idx_item_id, table_item_id,
           idx_device, table_device, idx_geo, table_geo,
           idx_hour, table_hour, idx_dayofweek, table_dayofweek,
           W0, b0, W1, b1, W2, b2):
  p0, p1 = _tc_transpose(table_user_id.T, table_item_id.T)
  tables = [p0.reshape(_VPAD, D), p1.reshape(_VPAD, D), table_device,
            table_geo, table_hour, table_dayofweek]
  i0 = _pack_idx(idx_user_id.astype(jnp.int32))
  i1 = _pack_idx(idx_item_id.astype(jnp.int32))
  indices = (i0, i1) + tuple(
      i.astype(jnp.int32)
      for i in (idx_device, idx_geo, idx_hour, idx_dayofweek))
  embs = _sc_gather(tables, indices)
  return _tc_mlp(embs, W0, b0, W1, b1, W2, b2)
